# baseline (speedup 1.0000x reference)
import jax
import jax.numpy as jnp
from jax import lax
from jax.experimental import pallas as pl
from jax.experimental.pallas import tpu as pltpu
from jax.experimental.pallas import tpu_sc as plsc

V = 100000
D = 64
B = 16384
L = 16
NC, NS = 2, 16
NSUB = 2
RANGE = 1568
RP = RANGE // 2
LCHUNK = 2048
NLCH = B // LCHUNK
GB = 32
DCH = D // L


def _body(feat_hbm, lab_hbm, cent_hbm, beta_hbm, out_hbm,
          cstage, lab_v, occ_l, loc_l, fstage, gidx, cnt, touched, beta_v,
          sem_in, sem_out, sem_g):
    wid = lax.axis_index("s") * NC + lax.axis_index("c")
    iota = lax.broadcasted_iota(jnp.int32, (L,), 0)

    pltpu.sync_copy(beta_hbm, beta_v)
    beta = beta_v[...]
    nbeta = -beta

    def _zocc(i, carry):
        occ_l[pl.ds(i * L, L)] = jnp.zeros((L,), jnp.int32)
        return carry
    lax.fori_loop(0, (LCHUNK + L) // L, _zocc, 0)

    for s in range(NSUB):
        base = jnp.minimum((wid * NSUB + s) * RANGE, V - RANGE)
        pbase = pl.multiple_of(base // 2, 8)
        h_in = pltpu.async_copy(cent_hbm.at[pl.ds(pbase, RP)], cstage, sem_in)

        def _zcnt(i, carry):
            cnt[pl.ds(i * L, L)] = jnp.zeros((L,), jnp.float32)
            return carry
        lax.fori_loop(0, RANGE // L, _zcnt, 0)
        h_in.wait()

        for c in range(NLCH):
            pltpu.sync_copy(lab_hbm.at[pl.ds(c * LCHUNK, LCHUNK)], lab_v)

            def _scan(i, m, c=c):
                lv = lab_v[pl.ds(i * L, L)]
                msk = (lv >= base) & (lv < base + RANGE)
                pos = m + plsc.cumsum(msk.astype(jnp.int32)) - 1
                occ_v = iota + (c * LCHUNK + i * L)
                loc_v = lv - base
                plsc.store_scatter(occ_l, [pos], occ_v, mask=msk)
                plsc.store_scatter(loc_l, [pos], loc_v, mask=msk)
                return m + jnp.sum(msk.astype(jnp.int32))
            m = lax.fori_loop(0, LCHUNK // L, _scan, jnp.int32(0))

            nb = (m + GB - 1) // GB

            def _flush(g, carry, m=m):
                gs = g * GB
                for i in range(GB // L):
                    ov = occ_l[pl.ds(gs + i * L, L)]
                    gidx[pl.ds(i * L, L)] = lax.shift_right_logical(ov, 1)
                pltpu.async_copy(feat_hbm.at[gidx], fstage, sem_g).wait()
                kcount = jnp.minimum(GB, m - gs)

                def _occ(k, carry2, gs=gs):
                    o = occ_l[pl.ds(gs + k, L)][0]
                    r = loc_l[pl.ds(gs + k, L)][0]
                    f_off = (o & 1) * D
                    c_off = (r & 1) * D
                    p2 = lax.shift_right_logical(r, 1)
                    for j in range(DCH):
                        cv = cstage[p2, pl.ds(c_off + j * L, L)]
                        fv = fstage[k, pl.ds(f_off + j * L, L)]
                        cstage[p2, pl.ds(c_off + j * L, L)] = cv + nbeta * fv
                    b16 = (r // L) * L
                    oh = jnp.where(iota == (r - b16),
                                   jnp.float32(1.0), jnp.float32(0.0))
                    plsc.addupdate(cnt.at[pl.ds(b16, L)], oh)
                    return carry2
                lax.fori_loop(0, kcount, _occ, 0)
                return carry
            lax.fori_loop(0, nb, _flush, 0)

        def _tscan(i, t):
            cv = cnt[pl.ds(i * L, L)]
            msk = cv > 0.0
            pos = t + plsc.cumsum(msk.astype(jnp.int32)) - 1
            plsc.store_scatter(touched, [pos], iota + i * L, mask=msk)
            return t + jnp.sum(msk.astype(jnp.int32))
        t = lax.fori_loop(0, RANGE // L, _tscan, jnp.int32(0))

        for i in range(GB // L):
            touched[pl.ds(t + i * L, L)] = jnp.zeros((L,), jnp.int32)

        nbt = (t + GB - 1) // GB

        def _tflush(g, carry, t=t, pbase=pbase):
            gs = g * GB
            for i in range(GB // L):
                tv = touched[pl.ds(gs + i * L, L)]
                gidx[pl.ds(i * L, L)] = pbase + lax.shift_right_logical(tv, 1)
            pltpu.async_copy(cent_hbm.at[gidx], fstage, sem_g).wait()
            kcount = jnp.minimum(GB, t - gs)

            def _tocc(k, carry2, gs=gs):
                r = touched[pl.ds(gs + k, L)][0]
                csplat = plsc.load_gather(cnt, [jnp.full((L,), r, jnp.int32)])
                scale = beta * csplat
                c_off = (r & 1) * D
                p2 = lax.shift_right_logical(r, 1)
                for j in range(DCH):
                    cv = cstage[p2, pl.ds(c_off + j * L, L)]
                    ov = fstage[k, pl.ds(c_off + j * L, L)]
                    cstage[p2, pl.ds(c_off + j * L, L)] = cv + scale * ov
                return carry2
            lax.fori_loop(0, kcount, _tocc, 0)
            return carry
        lax.fori_loop(0, nbt, _tflush, 0)

        h_out = pltpu.async_copy(cstage, out_hbm.at[pl.ds(pbase, RP)], sem_out)
        h_out.wait()


def kernel(features, labels, centers, alpha):
    beta16 = jnp.full((L,), alpha, jnp.float32) - 1.0
    labels = labels.astype(jnp.int32)
    feat2 = features.reshape(B // 2, 2 * D)
    cent2 = centers.reshape(V // 2, 2 * D)
    run = pl.kernel(
        _body,
        out_type=jax.ShapeDtypeStruct((V // 2, 2 * D), jnp.float32),
        mesh=plsc.VectorSubcoreMesh(core_axis_name="c", subcore_axis_name="s",
                                    num_cores=NC, num_subcores=NS),
        scratch_types=[
            pltpu.VMEM((RP, 2 * D), jnp.float32),
            pltpu.VMEM((LCHUNK,), jnp.int32),
            pltpu.VMEM((LCHUNK + L,), jnp.int32),
            pltpu.VMEM((LCHUNK + L,), jnp.int32),
            pltpu.VMEM((GB, 2 * D), jnp.float32),
            pltpu.VMEM((GB,), jnp.int32),
            pltpu.VMEM((RANGE,), jnp.float32),
            pltpu.VMEM((RANGE + GB + L,), jnp.int32),
            pltpu.VMEM((L,), jnp.float32),
            pltpu.SemaphoreType.DMA,
            pltpu.SemaphoreType.DMA,
            pltpu.SemaphoreType.DMA,
        ],
        compiler_params=pltpu.CompilerParams(needs_layout_passes=False),
    )
    out2 = run(feat2, labels, cent2, beta16)
    return out2.reshape(V, D)
